# Initial kernel scaffold; baseline (speedup 1.0000x reference)
import jax
import jax.numpy as jnp
from jax.experimental import pallas as pl


def kernel(x, edge_index, edge_attr, xe1, xe2, ee1, ee2, W1, b1, W2, b2, gamma, beta):
    raise NotImplementedError("write your pallas kernel here")


# speedup vs baseline: 1.3568x; 1.3568x over previous
import functools

import jax
import jax.numpy as jnp
import numpy as np
from jax import lax
from jax.experimental import pallas as pl
from jax.experimental.pallas import tpu as pltpu
from jax.experimental.pallas import tpu_sc as plsc

N = 10000
E = 320000
EF = E + N
EMB = 128
NLAYERS = 5
NC, NS = 2, 16
NW = NC * NS
CHUNK = 128
NCHUNK = -(-EF // (NW * CHUNK))
EW = NCHUNK * CHUNK
EPAD = EW * NW


@functools.lru_cache(maxsize=None)
def _sc_gather():
    mesh = plsc.VectorSubcoreMesh(core_axis_name="c", subcore_axis_name="s")

    @functools.partial(
        pl.kernel,
        out_type=jax.ShapeDtypeStruct((NW, EW, EMB), jnp.float32),
        mesh=mesh,
        scratch_types=[
            pltpu.VMEM((NCHUNK, CHUNK), jnp.int32),
            pltpu.VMEM((CHUNK, EMB), jnp.float32),
            pltpu.SemaphoreType.DMA,
        ],
    )
    def body(msg_hbm, gidx_hbm, out_hbm, idxv, rows, sem):
        c = lax.axis_index("c")
        s = lax.axis_index("s")
        wid = c * NS + s
        pltpu.sync_copy(gidx_hbm.at[wid], idxv)

        @pl.loop(0, NCHUNK)
        def _(i):
            pltpu.async_copy(msg_hbm.at[idxv.at[i]], rows, sem).wait()
            pltpu.sync_copy(rows, out_hbm.at[wid, pl.ds(i * CHUNK, CHUNK)])

    return body


def kernel(x, edge_index, edge_attr, xe1, xe2, ee1, ee2, W1, b1, W2, b2,
           gamma, beta):
    n = N
    h = xe1[x[:, 0]] + xe2[x[:, 1]]

    loop = jnp.arange(n, dtype=edge_index.dtype)
    src_f = jnp.concatenate([edge_index[0], loop])
    dst_f = jnp.concatenate([edge_index[1], loop])
    cid_f = jnp.concatenate([edge_attr[:, 0] * 3 + edge_attr[:, 1],
                             jnp.full((n,), 9, jnp.int32)])
    perm = jnp.argsort(dst_f, stable=True)
    ss, ds, cs = src_f[perm], dst_f[perm], cid_f[perm]

    npad = EPAD - EF
    gidx = (cs * n + ss)
    gidx = jnp.concatenate([gidx, jnp.zeros((npad,), jnp.int32)]).reshape(
        NW, NCHUNK, CHUNK)

    i1 = jnp.asarray([0, 0, 0, 1, 1, 1, 2, 2, 2, 4])
    i2 = jnp.asarray([0, 1, 2, 0, 1, 2, 0, 1, 2, 0])

    sc_call = _sc_gather()
    for l in range(NLAYERS):
        ctab = ee1[l][i1] + ee2[l][i2]
        msgtab = (ctab[:, None, :] + h[None, :, :]).reshape(10 * n, EMB)
        msg = sc_call(msgtab, gidx).reshape(EPAD, EMB)[:EF]
        agg = jax.ops.segment_sum(msg, ds, num_segments=n)
        hid = jax.nn.relu(agg @ W1[l] + b1[l])
        out = hid @ W2[l] + b2[l]
        mean = jnp.mean(out, axis=0)
        var = jnp.var(out, axis=0)
        out = gamma[l] * (out - mean) / jnp.sqrt(var + 1e-5) + beta[l]
        if l != NLAYERS - 1:
            out = jax.nn.relu(out)
        h = out
    return h
